# baseline (speedup 1.0000x reference)
import functools

import jax
import jax.numpy as jnp
from jax import lax
from jax.experimental import pallas as pl
from jax.experimental.pallas import tpu as pltpu
from jax.experimental.pallas import tpu_sc as plsc

N = 10000
E = 320000
D = 128

NTILES = 16
NCORES = 2
K = 64
NB = 160
NBUF = 4
SB = 16
NSTAGE = NB // SB
EDGES_PER_TILE = NB * K
E_PAD = NCORES * NTILES * EDGES_PER_TILE
N_PAD = 10240
ROWS_PER_TILE = N_PAD // NTILES

_GDN = lax.GatherDimensionNumbers(
    offset_dims=(), collapsed_slice_dims=(0,), start_index_map=(0,))


def _mm_body(x_ref, w_ref, o_ref):
    o_ref[...] = jnp.dot(x_ref[...], w_ref[...],
                         preferred_element_type=jnp.float32)


def _matmul(x, W):
    return pl.pallas_call(
        _mm_body,
        grid=(10,),
        in_specs=[
            pl.BlockSpec((N // 10, D), lambda r: (r, 0)),
            pl.BlockSpec((D, D), lambda r: (0, 0)),
        ],
        out_specs=pl.BlockSpec((N // 10, D), lambda r: (r, 0)),
        out_shape=jax.ShapeDtypeStruct((N, D), jnp.float32),
    )(x, W)


def _sc_body(h_hbm, src_hbm, dst_hbm, w_hbm, out_hbm,
             src_v, dst_v, w_v, rows_v, acc,
             sg0, sg1, sg2, sg3, ss0, ss1, ss2, ss3, sp0, sp1, sp2):
    c = lax.axis_index("c")
    s = lax.axis_index("s")
    chunk = c * NTILES + s
    sg = (sg0, sg1, sg2, sg3)
    ss = (ss0, ss1, ss2, ss3)

    def issue_prefetch(stage, p):
        row0 = stage * SB
        ele0 = stage * (SB * K)
        pltpu.async_copy(src_hbm.at[chunk].at[pl.ds(row0, SB)],
                         src_v.at[p], sp0)
        pltpu.async_copy(dst_hbm.at[chunk].at[pl.ds(row0, SB)],
                         dst_v.at[p], sp1)
        pltpu.async_copy(w_hbm.at[chunk].at[pl.ds(ele0, SB * K)],
                         w_v.at[p], sp2)

    def wait_prefetch(p):
        pltpu.make_async_copy(src_hbm.at[chunk].at[pl.ds(0, SB)],
                              src_v.at[p], sp0).wait()
        pltpu.make_async_copy(dst_hbm.at[chunk].at[pl.ds(0, SB)],
                              dst_v.at[p], sp1).wait()
        pltpu.make_async_copy(w_hbm.at[chunk].at[pl.ds(0, SB * K)],
                              w_v.at[p], sp2).wait()

    def issue_gather(p, lb, j):
        pltpu.async_copy(h_hbm.at[src_v.at[p].at[lb]], rows_v.at[j], sg[j])

    def wait_gather(p, lb, j):
        pltpu.make_async_copy(h_hbm.at[src_v.at[p].at[lb]],
                              rows_v.at[j], sg[j]).wait()

    def issue_scatter(p, lb, j):
        pltpu.async_copy(rows_v.at[j], acc.at[dst_v.at[p].at[lb]], ss[j],
                         add=True)

    def wait_scatter(p, lb, j):
        pltpu.make_async_copy(rows_v.at[j], acc.at[dst_v.at[p].at[lb]],
                              ss[j]).wait()

    def scale(j, lb, p):
        base = lb * K
        wref = w_v.at[p]
        for g in range(K // 16):
            wrow = wref[pl.ds(base + g * 16, 16)]
            for i in range(16):
                wv = wrow[i]
                r = g * 16 + i
                for k in range(D // 16):
                    rows_v[j, r, pl.ds(k * 16, 16)] = (
                        rows_v[j, r, pl.ds(k * 16, 16)] * wv)

    def batch_step(p, g4, u, stage, first_group=False):
        lb = g4 * 4 + u
        wait_gather(p, lb, u)
        j = (u + 2) % 4
        if first_group and u < 2:
            issue_gather(p, lb + 2, j)
        elif u < 2:
            wait_scatter(p, lb, j)
            issue_gather(p, lb + 2, j)
        elif isinstance(g4, int):
            if g4 < 3:
                wait_scatter(p, lb, j)
                issue_gather(p, lb + 2, j)
        else:
            @pl.when(g4 < 3)
            def _():
                wait_scatter(p, lb, j)
                issue_gather(p, lb + 2, j)
        scale(u, lb, p)
        issue_scatter(p, lb, u)
        if u == 1 and not first_group:
            @pl.when(jnp.logical_and(g4 == 0, stage < NSTAGE - 1))
            def _():
                issue_prefetch(stage + 1, 1 - p)

    issue_prefetch(0, 0)
    for k in range(D // 16):
        rows_v[0, 0, pl.ds(k * 16, 16)] = jnp.zeros((16,), jnp.float32)
    base = s * ROWS_PER_TILE
    pltpu.sync_copy(rows_v.at[0].at[pl.ds(0, 1)], acc.at[pl.ds(base, 1)])
    filled = 1
    while filled < ROWS_PER_TILE:
        n = min(filled, ROWS_PER_TILE - filled)
        pltpu.sync_copy(acc.at[pl.ds(base, n)],
                        acc.at[pl.ds(base + filled, n)])
        filled += n
    plsc.subcore_barrier()
    wait_prefetch(0)

    issue_gather(0, 0, 0)
    issue_gather(0, 1, 1)
    for u in range(4):
        batch_step(0, 0, u, 0, first_group=True)
    issue_prefetch(1, 1)

    def group_body_for(p, stage):
        def body(g4, carry):
            for u in range(4):
                batch_step(p, g4, u, stage)
            return carry
        return body

    lax.fori_loop(1, 4, group_body_for(0, jnp.int32(0)), 0)

    def stage_body(stage, carry):
        p = lax.rem(stage, 2)
        wait_prefetch(p)
        wait_scatter(p, 0, 0)
        issue_gather(p, 0, 0)
        wait_scatter(p, 1, 1)
        issue_gather(p, 1, 1)
        lax.fori_loop(0, 4, group_body_for(p, stage), 0)
        return carry

    lax.fori_loop(1, NSTAGE, stage_body, 0)

    for u in range(4):
        wait_scatter(1, 12 + u, u)

    plsc.subcore_barrier()
    pltpu.sync_copy(acc.at[pl.ds(s * ROWS_PER_TILE, ROWS_PER_TILE)],
                    out_hbm.at[c].at[pl.ds(s * ROWS_PER_TILE, ROWS_PER_TILE)])


_sc_scatter = functools.partial(
    pl.kernel,
    out_type=jax.ShapeDtypeStruct((NCORES, N_PAD, D), jnp.float32),
    mesh=plsc.VectorSubcoreMesh(core_axis_name="c", subcore_axis_name="s"),
    scratch_types=[
        pltpu.VMEM((2, SB, K), jnp.int32),
        pltpu.VMEM((2, SB, K), jnp.int32),
        pltpu.VMEM((2, SB * K), jnp.float32),
        pltpu.VMEM((NBUF, K, D), jnp.float32),
        pltpu.VMEM_SHARED((N_PAD, D), jnp.float32),
        pltpu.SemaphoreType.DMA,
        pltpu.SemaphoreType.DMA,
        pltpu.SemaphoreType.DMA,
        pltpu.SemaphoreType.DMA,
        pltpu.SemaphoreType.DMA,
        pltpu.SemaphoreType.DMA,
        pltpu.SemaphoreType.DMA,
        pltpu.SemaphoreType.DMA,
        pltpu.SemaphoreType.DMA,
        pltpu.SemaphoreType.DMA,
        pltpu.SemaphoreType.DMA,
    ],
)(_sc_body)


def _comb_body(p_ref, b_ref, o_ref):
    o_ref[...] = p_ref[0] + p_ref[1] + b_ref[...]


def _combine(parts, b2d):
    return pl.pallas_call(
        _comb_body,
        grid=(10,),
        in_specs=[
            pl.BlockSpec((NCORES, N // 10, D), lambda r: (0, r, 0)),
            pl.BlockSpec((1, D), lambda r: (0, 0)),
        ],
        out_specs=pl.BlockSpec((N // 10, D), lambda r: (r, 0)),
        out_shape=jax.ShapeDtypeStruct((N, D), jnp.float32),
    )(parts, b2d)


def kernel(x, edge_index, edge_weight, W, b):
    npad = E_PAD - E
    pad_src = (jnp.arange(npad, dtype=jnp.int32) % N)
    pad_dst = (jnp.arange(npad, dtype=jnp.int32) % N_PAD)
    src = jnp.concatenate([edge_index[1].astype(jnp.int32), pad_src])
    dst = jnp.concatenate([edge_index[0].astype(jnp.int32), pad_dst])
    w = jnp.concatenate(
        [edge_weight.astype(jnp.float32), jnp.zeros((npad,), jnp.float32)])
    src = src.reshape(NCORES * NTILES, NB, K)
    dst = dst.reshape(NCORES * NTILES, NB, K)
    w = w.reshape(NCORES * NTILES, NB * K)
    h = _matmul(x.astype(jnp.float32), W.astype(jnp.float32))
    parts = _sc_scatter(h, src, dst, w)
    return _combine(parts, b.astype(jnp.float32).reshape(1, D))


# speedup vs baseline: 2.2473x; 2.2473x over previous
import functools

import jax
import jax.numpy as jnp
from jax import lax
from jax.experimental import pallas as pl
from jax.experimental.pallas import tpu as pltpu
from jax.experimental.pallas import tpu_sc as plsc

N = 10000
E = 320000
D = 128

NTILES = 16
NCORES = 2
K = 64
NB = 160
NBUF = 4
SB = 16
NSTAGE = NB // SB
EDGES_PER_TILE = NB * K
E_PAD = NCORES * NTILES * EDGES_PER_TILE
N_PAD = 10240
ROWS_PER_TILE = N_PAD // NTILES

_GDN = lax.GatherDimensionNumbers(
    offset_dims=(), collapsed_slice_dims=(0,), start_index_map=(0,))


def _mm_body(x_ref, w_ref, o_ref):
    o_ref[...] = jnp.dot(x_ref[...], w_ref[...],
                         preferred_element_type=jnp.float32)


def _matmul(x, W):
    return pl.pallas_call(
        _mm_body,
        grid=(10,),
        in_specs=[
            pl.BlockSpec((N // 10, D), lambda r: (r, 0)),
            pl.BlockSpec((D, D), lambda r: (0, 0)),
        ],
        out_specs=pl.BlockSpec((N // 10, D), lambda r: (r, 0)),
        out_shape=jax.ShapeDtypeStruct((N, D), jnp.float32),
    )(x, W)


def _sc_body(h_hbm, src_hbm, dst_hbm, w_hbm, z_hbm, out_hbm,
             src_v, dst_v, w_v, rows_v, acc,
             sg0, sg1, sg2, sg3, ss0, ss1, ss2, ss3, sp0, sp1, sp2):
    c = lax.axis_index("c")
    s = lax.axis_index("s")
    chunk = c * NTILES + s
    sg = (sg0, sg1, sg2, sg3)
    ss = (ss0, ss1, ss2, ss3)

    def issue_prefetch(stage, p):
        row0 = stage * SB
        ele0 = stage * (SB * K)
        pltpu.async_copy(src_hbm.at[chunk].at[pl.ds(row0, SB)],
                         src_v.at[p], sp0)
        pltpu.async_copy(dst_hbm.at[chunk].at[pl.ds(row0, SB)],
                         dst_v.at[p], sp1)
        pltpu.async_copy(w_hbm.at[chunk].at[pl.ds(ele0, SB * K)],
                         w_v.at[p], sp2)

    def wait_prefetch(p):
        pltpu.make_async_copy(src_hbm.at[chunk].at[pl.ds(0, SB)],
                              src_v.at[p], sp0).wait()
        pltpu.make_async_copy(dst_hbm.at[chunk].at[pl.ds(0, SB)],
                              dst_v.at[p], sp1).wait()
        pltpu.make_async_copy(w_hbm.at[chunk].at[pl.ds(0, SB * K)],
                              w_v.at[p], sp2).wait()

    def issue_gather(p, lb, j):
        pltpu.async_copy(h_hbm.at[src_v.at[p].at[lb]], rows_v.at[j], sg[j])

    def wait_gather(p, lb, j):
        pltpu.make_async_copy(h_hbm.at[src_v.at[p].at[lb]],
                              rows_v.at[j], sg[j]).wait()

    def issue_scatter(p, lb, j):
        pltpu.async_copy(rows_v.at[j], acc.at[dst_v.at[p].at[lb]], ss[j],
                         add=True)

    def wait_scatter(p, lb, j):
        pltpu.make_async_copy(rows_v.at[j], acc.at[dst_v.at[p].at[lb]],
                              ss[j]).wait()

    def scale(j, lb, p):
        base = lb * K
        wref = w_v.at[p]
        for g in range(K // 16):
            wrow = wref[pl.ds(base + g * 16, 16)]
            for i in range(16):
                wv = wrow[i]
                r = g * 16 + i
                for k in range(D // 16):
                    rows_v[j, r, pl.ds(k * 16, 16)] = (
                        rows_v[j, r, pl.ds(k * 16, 16)] * wv)

    def batch_step(p, g4, u, stage, first_group=False):
        lb = g4 * 4 + u
        wait_gather(p, lb, u)
        j = (u + 2) % 4
        if first_group and u < 2:
            issue_gather(p, lb + 2, j)
        elif u < 2:
            wait_scatter(p, lb, j)
            issue_gather(p, lb + 2, j)
        elif isinstance(g4, int):
            if g4 < 3:
                wait_scatter(p, lb, j)
                issue_gather(p, lb + 2, j)
        else:
            @pl.when(g4 < 3)
            def _():
                wait_scatter(p, lb, j)
                issue_gather(p, lb + 2, j)
        scale(u, lb, p)
        issue_scatter(p, lb, u)
        if u == 1 and not first_group:
            @pl.when(jnp.logical_and(g4 == 0, stage < NSTAGE - 1))
            def _():
                issue_prefetch(stage + 1, 1 - p)

    issue_prefetch(0, 0)
    pltpu.sync_copy(z_hbm, acc.at[pl.ds(s * ROWS_PER_TILE, ROWS_PER_TILE)])
    plsc.subcore_barrier()
    wait_prefetch(0)

    issue_gather(0, 0, 0)
    issue_gather(0, 1, 1)
    for u in range(4):
        batch_step(0, 0, u, 0, first_group=True)
    issue_prefetch(1, 1)

    def group_body_for(p, stage):
        def body(g4, carry):
            for u in range(4):
                batch_step(p, g4, u, stage)
            return carry
        return body

    lax.fori_loop(1, 4, group_body_for(0, jnp.int32(0)), 0)

    def stage_body(stage, carry):
        p = lax.rem(stage, 2)
        wait_prefetch(p)
        wait_scatter(p, 0, 0)
        issue_gather(p, 0, 0)
        wait_scatter(p, 1, 1)
        issue_gather(p, 1, 1)
        lax.fori_loop(0, 4, group_body_for(p, stage), 0)
        return carry

    lax.fori_loop(1, NSTAGE, stage_body, 0)

    for u in range(4):
        wait_scatter(1, 12 + u, u)

    plsc.subcore_barrier()
    pltpu.sync_copy(acc.at[pl.ds(s * ROWS_PER_TILE, ROWS_PER_TILE)],
                    out_hbm.at[c].at[pl.ds(s * ROWS_PER_TILE, ROWS_PER_TILE)])


_sc_scatter = functools.partial(
    pl.kernel,
    out_type=jax.ShapeDtypeStruct((NCORES, N_PAD, D), jnp.float32),
    mesh=plsc.VectorSubcoreMesh(core_axis_name="c", subcore_axis_name="s"),
    scratch_types=[
        pltpu.VMEM((2, SB, K), jnp.int32),
        pltpu.VMEM((2, SB, K), jnp.int32),
        pltpu.VMEM((2, SB * K), jnp.float32),
        pltpu.VMEM((NBUF, K, D), jnp.float32),
        pltpu.VMEM_SHARED((N_PAD, D), jnp.float32),
        pltpu.SemaphoreType.DMA,
        pltpu.SemaphoreType.DMA,
        pltpu.SemaphoreType.DMA,
        pltpu.SemaphoreType.DMA,
        pltpu.SemaphoreType.DMA,
        pltpu.SemaphoreType.DMA,
        pltpu.SemaphoreType.DMA,
        pltpu.SemaphoreType.DMA,
        pltpu.SemaphoreType.DMA,
        pltpu.SemaphoreType.DMA,
        pltpu.SemaphoreType.DMA,
    ],
)(_sc_body)


def _comb_body(p_ref, b_ref, o_ref):
    o_ref[...] = p_ref[0] + p_ref[1] + b_ref[...]


def _combine(parts, b2d):
    return pl.pallas_call(
        _comb_body,
        grid=(10,),
        in_specs=[
            pl.BlockSpec((NCORES, N // 10, D), lambda r: (0, r, 0)),
            pl.BlockSpec((1, D), lambda r: (0, 0)),
        ],
        out_specs=pl.BlockSpec((N // 10, D), lambda r: (r, 0)),
        out_shape=jax.ShapeDtypeStruct((N, D), jnp.float32),
    )(parts, b2d)


def kernel(x, edge_index, edge_weight, W, b):
    npad = E_PAD - E
    pad_src = (jnp.arange(npad, dtype=jnp.int32) % N)
    pad_dst = (jnp.arange(npad, dtype=jnp.int32) % N_PAD)
    src = jnp.concatenate([edge_index[1].astype(jnp.int32), pad_src])
    dst = jnp.concatenate([edge_index[0].astype(jnp.int32), pad_dst])
    w = jnp.concatenate(
        [edge_weight.astype(jnp.float32), jnp.zeros((npad,), jnp.float32)])
    src = src.reshape(NCORES * NTILES, NB, K)
    dst = dst.reshape(NCORES * NTILES, NB, K)
    w = w.reshape(NCORES * NTILES, NB * K)
    z = jnp.zeros((ROWS_PER_TILE, D), jnp.float32)
    h = _matmul(x.astype(jnp.float32), W.astype(jnp.float32))
    parts = _sc_scatter(h, src, dst, w, z)
    return _combine(parts, b.astype(jnp.float32).reshape(1, D))
